# Initial kernel scaffold; baseline (speedup 1.0000x reference)
import jax
import jax.numpy as jnp
from jax.experimental import pallas as pl


def kernel(queries, keys):
    raise NotImplementedError("write your pallas kernel here")


# speedup vs baseline: 4.6039x; 4.6039x over previous
import functools

import jax
import jax.numpy as jnp
from jax.experimental import pallas as pl
from jax.experimental.pallas import tpu as pltpu

KB = 2048
BIG_F = 1.0e37
BIG_I = 2 ** 30

DOT_PRECISION = jax.lax.Precision.DEFAULT


def _knn_kernel(q_ref, k_ref, score_ref, idx_ref, rv_ref, ri_ref, *, nsteps, nkeys):
    pid = pl.program_id(0)

    @pl.when(pid == 0)
    def _init():
        rv_ref[...] = jnp.full_like(rv_ref, -3.0e38)
        ri_ref[...] = jnp.zeros_like(ri_ref)

    kb = k_ref[...]
    base = pid * KB
    row_id = jax.lax.broadcasted_iota(jnp.int32, kb.shape, 0) + base
    kb = jnp.where(row_id < nkeys, kb, 0.0)

    q = q_ref[...]
    dot = jax.lax.dot_general(
        q, kb, (((1,), (1,)), ((), ())),
        preferred_element_type=jnp.float32,
        precision=DOT_PRECISION,
    )

    hk = 0.5 * jnp.sum(kb * kb, axis=1)
    col_id = jax.lax.broadcasted_iota(jnp.int32, (1, KB), 1) + base
    hk = jnp.where(col_id < nkeys, hk.reshape(1, KB), BIG_F)

    s = dot - hk
    bmax = jnp.max(s, axis=1, keepdims=True)
    cand = jnp.min(
        jnp.where(s == bmax, col_id, BIG_I), axis=1, keepdims=True
    ).astype(jnp.int32)

    upd = bmax > rv_ref[...]
    rv_ref[...] = jnp.where(upd, bmax, rv_ref[...])
    ri_ref[...] = jnp.where(upd, cand, ri_ref[...])

    @pl.when(pid == nsteps - 1)
    def _emit():
        q_sq = jnp.sum(q * q, axis=1, keepdims=True)
        d2 = jnp.maximum(q_sq - 2.0 * rv_ref[...], 0.0)
        score_ref[...] = jnp.sqrt(d2)
        idx_ref[...] = ri_ref[...]


def kernel(queries, keys):
    nq, d = queries.shape
    nkeys = keys.shape[0]
    nsteps = pl.cdiv(nkeys, KB)

    score, idx = pl.pallas_call(
        functools.partial(_knn_kernel, nsteps=nsteps, nkeys=nkeys),
        grid=(nsteps,),
        in_specs=[
            pl.BlockSpec((nq, d), lambda i: (0, 0)),
            pl.BlockSpec((KB, d), lambda i: (i, 0)),
        ],
        out_specs=[
            pl.BlockSpec((nq, 1), lambda i: (0, 0)),
            pl.BlockSpec((nq, 1), lambda i: (0, 0)),
        ],
        out_shape=[
            jax.ShapeDtypeStruct((nq, 1), jnp.float32),
            jax.ShapeDtypeStruct((nq, 1), jnp.int32),
        ],
        scratch_shapes=[
            pltpu.VMEM((nq, 1), jnp.float32),
            pltpu.VMEM((nq, 1), jnp.int32),
        ],
    )(queries, keys)

    return score.reshape(nq), idx


# speedup vs baseline: 6.4861x; 1.4088x over previous
import functools

import jax
import jax.numpy as jnp
from jax.experimental import pallas as pl
from jax.experimental.pallas import tpu as pltpu

KB = 2048
BIG_F = 1.0e37
BIG_I = 2 ** 30

DOT_PRECISION = jax.lax.Precision.DEFAULT


def _knn_kernel(q_ref, k_ref, score_ref, idx_ref, rv_ref, ri_ref, *, nsteps, nkeys):
    pid = pl.program_id(0)

    @pl.when(pid == 0)
    def _init():
        rv_ref[...] = jnp.full_like(rv_ref, -3.0e38)
        ri_ref[...] = jnp.zeros_like(ri_ref)

    kb = k_ref[...]
    base = pid * KB
    row_id = jax.lax.broadcasted_iota(jnp.int32, kb.shape, 0) + base
    kb = jnp.where(row_id < nkeys, kb, 0.0)

    q = q_ref[...]
    dot = jax.lax.dot_general(
        q, kb, (((1,), (1,)), ((), ())),
        preferred_element_type=jnp.float32,
        precision=DOT_PRECISION,
    )

    hk = 0.5 * jnp.sum(kb * kb, axis=1)
    col_id = jax.lax.broadcasted_iota(jnp.int32, (1, KB), 1) + base
    hk = jnp.where(col_id < nkeys, hk.reshape(1, KB), BIG_F)

    s = dot - hk

    val = s[:, 0:128]
    idx = col_id[:, 0:128]
    for c in range(1, KB // 128):
        sc = s[:, c * 128:(c + 1) * 128]
        cc = col_id[:, c * 128:(c + 1) * 128]
        upd = sc > val
        val = jnp.where(upd, sc, val)
        idx = jnp.where(upd, cc, idx)

    upd = val > rv_ref[...]
    rv_ref[...] = jnp.where(upd, val, rv_ref[...])
    ri_ref[...] = jnp.where(upd, idx, ri_ref[...])

    @pl.when(pid == nsteps - 1)
    def _emit():
        av = rv_ref[...]
        ai = ri_ref[...]
        bmax = jnp.max(av, axis=1, keepdims=True)
        best = jnp.min(
            jnp.where(av == bmax, ai, BIG_I), axis=1, keepdims=True
        ).astype(jnp.int32)
        q_sq = jnp.sum(q * q, axis=1, keepdims=True)
        d2 = jnp.maximum(q_sq - 2.0 * bmax, 0.0)
        score_ref[...] = jnp.sqrt(d2)
        idx_ref[...] = best


def kernel(queries, keys):
    nq, d = queries.shape
    nkeys = keys.shape[0]
    nsteps = pl.cdiv(nkeys, KB)

    score, idx = pl.pallas_call(
        functools.partial(_knn_kernel, nsteps=nsteps, nkeys=nkeys),
        grid=(nsteps,),
        in_specs=[
            pl.BlockSpec((nq, d), lambda i: (0, 0)),
            pl.BlockSpec((KB, d), lambda i: (i, 0)),
        ],
        out_specs=[
            pl.BlockSpec((nq, 1), lambda i: (0, 0)),
            pl.BlockSpec((nq, 1), lambda i: (0, 0)),
        ],
        out_shape=[
            jax.ShapeDtypeStruct((nq, 1), jnp.float32),
            jax.ShapeDtypeStruct((nq, 1), jnp.int32),
        ],
        scratch_shapes=[
            pltpu.VMEM((nq, 128), jnp.float32),
            pltpu.VMEM((nq, 128), jnp.int32),
        ],
    )(queries, keys)

    return score.reshape(nq), idx
